# baseline (speedup 1.0000x reference)
import functools

import jax
import jax.numpy as jnp
from jax import lax
from jax.experimental import pallas as pl
from jax.experimental.pallas import tpu as pltpu
from jax.experimental.pallas import tpu_sc as plsc

_DIM = 128
_E = 320000
_V0, _V1, _V2 = 5, 6, 2
_NCOMB = _V0 * _V1 * _V2
_NC, _NS = 2, 16
_NW = _NC * _NS
_CHUNK = 256
_NCHUNK = _E // _CHUNK
_KMAX = -(-_NCHUNK // _NW)
_G16 = _CHUNK // 16
_TSTRIDE = 64

_mesh = plsc.VectorSubcoreMesh(core_axis_name="c", subcore_axis_name="s")


@functools.partial(
    pl.kernel,
    mesh=_mesh,
    out_type=(
        jax.ShapeDtypeStruct((_E, _DIM), jnp.float32),
        jax.ShapeDtypeStruct((_NC * _TSTRIDE, _DIM), jnp.float32),
    ),
    scratch_types=[
        pltpu.VMEM((_V0, _DIM), jnp.float32),
        pltpu.VMEM((_V1, _DIM), jnp.float32),
        pltpu.VMEM((_V2, _DIM), jnp.float32),
        pltpu.VMEM((_TSTRIDE, _DIM), jnp.float32),
        pltpu.VMEM((2, 3, _CHUNK), jnp.int32),
        pltpu.VMEM((2, 2, 128), jnp.int32),
        pltpu.VMEM((2, _CHUNK, _DIM), jnp.float32),
        pltpu.SemaphoreType.DMA,
        pltpu.SemaphoreType.DMA,
        pltpu.SemaphoreType.DMA,
        pltpu.SemaphoreType.DMA,
        pltpu.SemaphoreType.DMA,
        pltpu.SemaphoreType.DMA,
    ],
)
def _sc_encode(ef_hbm, w0_hbm, w1_hbm, w2_hbm, out_hbm, tab_hbm,
               w0_v, w1_v, w2_v, tab_v, ef_v, cb_v, rows_v,
               sem_ef0, sem_ef1, sem_g0, sem_g1, sem_out0, sem_out1):
    cid = lax.axis_index("c")
    sid = lax.axis_index("s")
    wid = sid * _NC + cid
    sem_ef = (sem_ef0, sem_ef1)
    sem_g = (sem_g0, sem_g1)
    sem_out = (sem_out0, sem_out1)

    def ef_slice(c):
        off = pl.multiple_of(c * _CHUNK, 128)
        return ef_hbm.at[:, pl.ds(off, _CHUNK)]

    def out_slice(c):
        off = pl.multiple_of(c * _CHUNK, 128)
        return out_hbm.at[pl.ds(off, _CHUNK)]

    pltpu.async_copy(ef_slice(wid), ef_v.at[0], sem_ef[0])

    pltpu.sync_copy(w0_hbm, w0_v)
    pltpu.sync_copy(w1_hbm, w1_v)
    pltpu.sync_copy(w2_hbm, w2_v)

    r0 = [[w0_v[i, pl.ds(j * 16, 16)] for j in range(8)] for i in range(_V0)]
    r1 = [[w1_v[i, pl.ds(j * 16, 16)] for j in range(8)] for i in range(_V1)]
    r2 = [[w2_v[i, pl.ds(j * 16, 16)] for j in range(8)] for i in range(_V2)]
    for c0 in range(_V0):
        for c1 in range(_V1):
            t01 = [r0[c0][j] + r1[c1][j] for j in range(8)]
            for c2 in range(_V2):
                row = (c0 * _V1 + c1) * _V2 + c2
                for j in range(8):
                    tab_v[row, pl.ds(j * 16, 16)] = t01[j] + r2[c2][j]

    @pl.when(sid == 0)
    def _publish():
        off = pl.multiple_of(cid * _TSTRIDE, 8)
        pltpu.sync_copy(tab_v, tab_hbm.at[pl.ds(off, _TSTRIDE)])

    plsc.subcore_barrier()

    tbase = cid * _TSTRIDE

    def pair_body(k2, carry):
        for b in range(2):
            k = k2 * 2 + b
            c = k * _NW + wid
            valid = c < _NCHUNK
            c_next = c + _NW

            @pl.when(valid)
            def _wait_ef(b=b, c=c):
                pltpu.make_async_copy(ef_slice(c), ef_v.at[b], sem_ef[b]).wait()

            @pl.when(c_next < _NCHUNK)
            def _issue_ef(b=b, c_next=c_next):
                pltpu.async_copy(ef_slice(c_next), ef_v.at[1 - b], sem_ef[1 - b])

            @pl.when(k2 >= 1)
            def _wait_out(b=b, c=c):
                pltpu.make_async_copy(
                    rows_v.at[b], out_slice(c - 2 * _NW), sem_out[b]).wait()

            @pl.when(valid)
            def _compute(b=b, c=c):
                for g in range(_G16):
                    e0 = ef_v[b, 0, pl.ds(g * 16, 16)]
                    e1 = ef_v[b, 1, pl.ds(g * 16, 16)]
                    e2 = ef_v[b, 2, pl.ds(g * 16, 16)]
                    comb = e0 * (_V1 * _V2) + e1 * _V2 + e2 + tbase
                    cb_v[b, g // 8, pl.ds((g % 8) * 16, 16)] = comb

                g0 = pltpu.async_copy(
                    tab_hbm.at[cb_v.at[b, 0]],
                    rows_v.at[b, pl.ds(0, 128)], sem_g[b])
                g1 = pltpu.async_copy(
                    tab_hbm.at[cb_v.at[b, 1]],
                    rows_v.at[b, pl.ds(128, 128)], sem_g[b])
                g0.wait()
                g1.wait()

                pltpu.async_copy(rows_v.at[b], out_slice(c), sem_out[b])

        return carry

    lax.fori_loop(0, _KMAX // 2, pair_body, 0)

    for b in range(2):
        c_last = (_KMAX - 2 + b) * _NW + wid

        @pl.when(c_last < _NCHUNK)
        def _drain(b=b, c_last=c_last):
            pltpu.make_async_copy(
                rows_v.at[b], out_slice(c_last), sem_out[b]).wait()


def kernel(edge_feat, W0, W1, W2):
    out, _ = _sc_encode(edge_feat.T, W0, W1, W2)
    return out


# speedup vs baseline: 3.2893x; 3.2893x over previous
import functools

import jax
import jax.numpy as jnp
from jax import lax
from jax.experimental import pallas as pl
from jax.experimental.pallas import tpu as pltpu
from jax.experimental.pallas import tpu_sc as plsc

_DIM = 128
_E = 320000
_V0, _V1, _V2 = 5, 6, 2
_NCOMB = _V0 * _V1 * _V2
_NC, _NS = 2, 16
_NW = _NC * _NS
_CHUNK = 256
_NCHUNK = _E // _CHUNK
_KMAX = -(-_NCHUNK // _NW)
_G16 = _CHUNK // 16

_mesh = plsc.VectorSubcoreMesh(core_axis_name="c", subcore_axis_name="s")


@functools.partial(
    pl.kernel,
    mesh=_mesh,
    out_type=jax.ShapeDtypeStruct((_E, _DIM), jnp.float32),
    scratch_types=[
        pltpu.VMEM((_V0, _DIM), jnp.float32),
        pltpu.VMEM((_V1, _DIM), jnp.float32),
        pltpu.VMEM((_V2, _DIM), jnp.float32),
        pltpu.VMEM((_NCOMB * _DIM,), jnp.float32),
        pltpu.VMEM((2, 3, _CHUNK), jnp.int32),
        pltpu.VMEM((_CHUNK + 16,), jnp.int32),
        pltpu.VMEM((2, _CHUNK, _DIM), jnp.float32),
        pltpu.SemaphoreType.DMA,
        pltpu.SemaphoreType.DMA,
        pltpu.SemaphoreType.DMA,
        pltpu.SemaphoreType.DMA,
    ],
)
def _sc_encode(ef_hbm, w0_hbm, w1_hbm, w2_hbm, out_hbm,
               w0_v, w1_v, w2_v, tab_v, ef_v, cb_v, out_v,
               sem_ef0, sem_ef1, sem_out0, sem_out1):
    wid = lax.axis_index("s") * _NC + lax.axis_index("c")
    sem_ef = (sem_ef0, sem_ef1)
    sem_out = (sem_out0, sem_out1)

    def ef_slice(c):
        off = pl.multiple_of(c * _CHUNK, 128)
        return ef_hbm.at[:, pl.ds(off, _CHUNK)]

    def out_slice(c):
        off = pl.multiple_of(c * _CHUNK, 128)
        return out_hbm.at[pl.ds(off, _CHUNK)]

    pltpu.async_copy(ef_slice(wid), ef_v.at[0], sem_ef[0])

    pltpu.sync_copy(w0_hbm, w0_v)
    pltpu.sync_copy(w1_hbm, w1_v)
    pltpu.sync_copy(w2_hbm, w2_v)

    r0 = [[w0_v[i, pl.ds(j * 16, 16)] for j in range(8)] for i in range(_V0)]
    r1 = [[w1_v[i, pl.ds(j * 16, 16)] for j in range(8)] for i in range(_V1)]
    r2 = [[w2_v[i, pl.ds(j * 16, 16)] for j in range(8)] for i in range(_V2)]
    for c0 in range(_V0):
        for c1 in range(_V1):
            t01 = [r0[c0][j] + r1[c1][j] for j in range(8)]
            for c2 in range(_V2):
                base = ((c0 * _V1 + c1) * _V2 + c2) * _DIM
                for j in range(8):
                    tab_v[pl.ds(base + j * 16, 16)] = t01[j] + r2[c2][j]

    def pair_body(k2, carry):
        for b in range(2):
            k = k2 * 2 + b
            c = k * _NW + wid
            valid = c < _NCHUNK
            c_next = c + _NW

            @pl.when(valid)
            def _wait_ef(b=b, c=c):
                pltpu.make_async_copy(ef_slice(c), ef_v.at[b], sem_ef[b]).wait()

            @pl.when(c_next < _NCHUNK)
            def _issue_ef(b=b, c_next=c_next):
                pltpu.async_copy(ef_slice(c_next), ef_v.at[1 - b], sem_ef[1 - b])

            @pl.when(k2 >= 1)
            def _wait_out(b=b, c=c):
                pltpu.make_async_copy(
                    out_v.at[b], out_slice(c - 2 * _NW), sem_out[b]).wait()

            @pl.when(valid)
            def _compute(b=b, c=c):
                for g in range(_G16):
                    e0 = ef_v[b, 0, pl.ds(g * 16, 16)]
                    e1 = ef_v[b, 1, pl.ds(g * 16, 16)]
                    e2 = ef_v[b, 2, pl.ds(g * 16, 16)]
                    comb = (e0 * (_V1 * _V2) + e1 * _V2 + e2) * _DIM
                    cb_v[pl.ds(g * 16, 16)] = comb

                def edge_body(e16, cc):
                    e = e16 * 16
                    bv = cb_v[pl.ds(e, 16)]
                    for u in range(16):
                        bb = bv[u]
                        for j in range(8):
                            out_v[b, e + u, pl.ds(j * 16, 16)] = (
                                tab_v[pl.ds(bb + j * 16, 16)])
                    return cc

                lax.fori_loop(0, _CHUNK // 16, edge_body, 0)

                pltpu.async_copy(out_v.at[b], out_slice(c), sem_out[b])

        return carry

    lax.fori_loop(0, _KMAX // 2, pair_body, 0)

    for b in range(2):
        c_last = (_KMAX - 2 + b) * _NW + wid

        @pl.when(c_last < _NCHUNK)
        def _drain(b=b, c_last=c_last):
            pltpu.make_async_copy(out_v.at[b], out_slice(c_last), sem_out[b]).wait()


def kernel(edge_feat, W0, W1, W2):
    return _sc_encode(edge_feat.T, W0, W1, W2)


# speedup vs baseline: 8.5985x; 2.6141x over previous
import functools

import jax
import jax.numpy as jnp
from jax import lax
from jax.experimental import pallas as pl
from jax.experimental.pallas import tpu as pltpu
from jax.experimental.pallas import tpu_sc as plsc

_DIM = 128
_E = 320000
_V0, _V1, _V2 = 5, 6, 2
_NCOMB = _V0 * _V1 * _V2
_NC, _NS = 2, 16
_NW = _NC * _NS
_CHUNK = 256
_NCHUNK = _E // _CHUNK
_KMAX = -(-_NCHUNK // _NW)
_G16 = _CHUNK // 16

_mesh = plsc.VectorSubcoreMesh(core_axis_name="c", subcore_axis_name="s")


@functools.partial(
    pl.kernel,
    mesh=_mesh,
    out_type=jax.ShapeDtypeStruct((_E, _DIM), jnp.float32),
    scratch_types=[
        pltpu.VMEM((_V0, _DIM), jnp.float32),
        pltpu.VMEM((_V1, _DIM), jnp.float32),
        pltpu.VMEM((_V2, _DIM), jnp.float32),
        pltpu.VMEM((_NCOMB * _DIM,), jnp.float32),
        pltpu.VMEM((2, 3, _CHUNK), jnp.int32),
        pltpu.VMEM((_CHUNK + 16,), jnp.int32),
        pltpu.VMEM((2, _CHUNK, _DIM), jnp.float32),
        pltpu.SemaphoreType.DMA,
        pltpu.SemaphoreType.DMA,
        pltpu.SemaphoreType.DMA,
        pltpu.SemaphoreType.DMA,
    ],
)
def _sc_encode(ef_hbm, w0_hbm, w1_hbm, w2_hbm, out_hbm,
               w0_v, w1_v, w2_v, tab_v, ef_v, cb_v, out_v,
               sem_ef0, sem_ef1, sem_out0, sem_out1):
    wid = lax.axis_index("s") * _NC + lax.axis_index("c")
    sem_ef = (sem_ef0, sem_ef1)
    sem_out = (sem_out0, sem_out1)

    def ef_slice(c):
        off = pl.multiple_of(c * _CHUNK, 128)
        return ef_hbm.at[:, pl.ds(off, _CHUNK)]

    def out_slice(c):
        off = pl.multiple_of(c * _CHUNK, 128)
        return out_hbm.at[pl.ds(off, _CHUNK)]

    pltpu.async_copy(ef_slice(wid), ef_v.at[0], sem_ef[0])

    pltpu.sync_copy(w0_hbm, w0_v)
    pltpu.sync_copy(w1_hbm, w1_v)
    pltpu.sync_copy(w2_hbm, w2_v)

    r0 = [[w0_v[i, pl.ds(j * 16, 16)] for j in range(8)] for i in range(_V0)]
    r1 = [[w1_v[i, pl.ds(j * 16, 16)] for j in range(8)] for i in range(_V1)]
    r2 = [[w2_v[i, pl.ds(j * 16, 16)] for j in range(8)] for i in range(_V2)]
    for c0 in range(_V0):
        for c1 in range(_V1):
            t01 = [r0[c0][j] + r1[c1][j] for j in range(8)]
            for c2 in range(_V2):
                base = ((c0 * _V1 + c1) * _V2 + c2) * _DIM
                for j in range(8):
                    tab_v[pl.ds(base + j * 16, 16)] = t01[j] + r2[c2][j]

    def pair_body(k2, carry):
        for b in range(2):
            k = k2 * 2 + b
            c = k * _NW + wid
            valid = c < _NCHUNK
            c_next = c + _NW

            @pl.when(valid)
            def _wait_ef(b=b, c=c):
                pltpu.make_async_copy(ef_slice(c), ef_v.at[b], sem_ef[b]).wait()

            @pl.when(c_next < _NCHUNK)
            def _issue_ef(b=b, c_next=c_next):
                pltpu.async_copy(ef_slice(c_next), ef_v.at[1 - b], sem_ef[1 - b])

            @pl.when(k2 >= 1)
            def _wait_out(b=b, c=c):
                pltpu.make_async_copy(
                    out_v.at[b], out_slice(c - 2 * _NW), sem_out[b]).wait()

            @pl.when(valid)
            def _compute(b=b, c=c):
                for g in range(_G16):
                    e0 = ef_v[b, 0, pl.ds(g * 16, 16)]
                    e1 = ef_v[b, 1, pl.ds(g * 16, 16)]
                    e2 = ef_v[b, 2, pl.ds(g * 16, 16)]
                    comb = (e0 * (_V1 * _V2) + e1 * _V2 + e2) * _DIM
                    cb_v[pl.ds(g * 16, 16)] = comb

                def edge_body(e16, cc):
                    e = e16 * 16
                    bv = cb_v[pl.ds(e, 16)]
                    for u in range(16):
                        bb = bv[u]
                        row = [tab_v[pl.ds(bb + j * 16, 16)] for j in range(8)]
                        for j in range(8):
                            out_v[b, e + u, pl.ds(j * 16, 16)] = row[j]
                    return cc

                lax.fori_loop(0, _CHUNK // 16, edge_body, 0)

                pltpu.async_copy(out_v.at[b], out_slice(c), sem_out[b])

        return carry

    lax.fori_loop(0, _KMAX // 2, pair_body, 0)

    for b in range(2):
        c_last = (_KMAX - 2 + b) * _NW + wid

        @pl.when(c_last < _NCHUNK)
        def _drain(b=b, c_last=c_last):
            pltpu.make_async_copy(out_v.at[b], out_slice(c_last), sem_out[b]).wait()


def kernel(edge_feat, W0, W1, W2):
    return _sc_encode(edge_feat.T, W0, W1, W2)
